# baseline (speedup 1.0000x reference)
import functools

import jax
import jax.numpy as jnp
from jax import lax
from jax.experimental import pallas as pl
from jax.experimental.pallas import tpu as pltpu
from jax.experimental.pallas import tpu_sc as plsc

_EMB = 128
_HALF = 64
_NG = 8192
_NA = 524288

_NC = 2
_NS = 16
_NW = _NC * _NS

_CHUNK = 64
_ROWS_PER_W = _NA // _NW
_NCHUNKS = _ROWS_PER_W // _CHUNK
_LEAD = 2


def _emb_body(t_ref, out_ref):
    t = t_ref[:, :]
    k = lax.broadcasted_iota(jnp.int32, (1, _EMB), 1).astype(jnp.float32)
    kmod = jnp.where(k < _HALF, k, k - _HALF)
    freqs = jnp.exp(-jnp.log(10000.0) * kmod / (_HALF - 1))
    phase = jnp.where(k < _HALF, 0.0, jnp.pi / 2.0)
    x = t * freqs + phase
    n = jnp.round(x * (1.0 / (2.0 * jnp.pi)))
    r = (x - n * 6.28125) - n * 1.9353071795864769e-3
    r2 = r * r
    p = 1.6059043836e-10
    for c in (-2.5052108385e-8, 2.7557319224e-6, -1.984126984e-4,
              8.3333333333e-3, -1.6666666666e-1):
        p = p * r2 + c
    out_ref[:, :] = r + r * (r2 * p)


def _emb_table(t_sampled):
    return pl.pallas_call(
        _emb_body,
        out_shape=jax.ShapeDtypeStruct((_NG, _EMB), jnp.float32),
    )(t_sampled)


_NBUF = 4
assert _NCHUNKS % _NBUF == 0
assert 0 < _LEAD < _NBUF


def _sc_gather_body(emb_hbm, batch_hbm, out_hbm, tbl_sh, idx_v, rows_v, gsem, ssem):
    sid = lax.axis_index("s")
    wid = sid * _NC + lax.axis_index("c")
    base = wid * _ROWS_PER_W

    @pl.when(sid == 0)
    def _():
        pltpu.sync_copy(emb_hbm, tbl_sh)

    pltpu.sync_copy(batch_hbm.at[wid], idx_v)
    plsc.subcore_barrier()

    def start_gather(j, b):
        pltpu.async_copy(tbl_sh.at[idx_v.at[j]], rows_v.at[b], gsem.at[b])

    def wait_gather(b):
        pltpu.make_async_copy(
            emb_hbm.at[pl.ds(0, _CHUNK)], rows_v.at[b], gsem.at[b]
        ).wait()

    def start_scatter(j, b):
        pltpu.async_copy(
            rows_v.at[b], out_hbm.at[pl.ds(base + j * _CHUNK, _CHUNK)], ssem.at[b]
        )

    def wait_scatter(b):
        pltpu.make_async_copy(
            emb_hbm.at[pl.ds(0, _CHUNK)], rows_v.at[b], ssem.at[b]
        ).wait()

    for j in range(_LEAD):
        start_gather(j, j)

    n_outer = _NCHUNKS // _NBUF

    def outer(go, carry):
        for b in range(_NBUF):
            g = go * _NBUF + b
            wait_gather(b)
            start_scatter(g, b)
            bg = (b + _LEAD) % _NBUF

            if b >= _NBUF - _LEAD:
                wait_scatter(bg)
            else:

                @pl.when(go > 0)
                def _():
                    wait_scatter(bg)

            if b < _NBUF - _LEAD:
                start_gather(g + _LEAD, bg)
            else:

                @pl.when(go < n_outer - 1)
                def _():
                    start_gather(g + _LEAD, bg)

        return carry

    lax.fori_loop(0, n_outer, outer, 0)

    for k in range(_NBUF - _LEAD):
        wait_scatter((_NCHUNKS - _NBUF + _LEAD + k) % _NBUF)


_sc_gather = functools.partial(
    pl.kernel,
    mesh=plsc.VectorSubcoreMesh(core_axis_name="c", subcore_axis_name="s"),
    out_type=jax.ShapeDtypeStruct((_NA, _EMB), jnp.float32),
    scratch_types=[
        pltpu.VMEM_SHARED((_NG, _EMB), jnp.float32),
        pltpu.VMEM((_NCHUNKS, _CHUNK), jnp.int32),
        pltpu.VMEM((_NBUF, _CHUNK, _EMB), jnp.float32),
        pltpu.SemaphoreType.DMA((_NBUF,)),
        pltpu.SemaphoreType.DMA((_NBUF,)),
    ],
)(_sc_gather_body)


_TC_BLK = 512
_TC_PAN = 128


def _tc_gather_body(lo_ref, hi_ref, batch_ref, emb_ref, out_ref):
    i = pl.program_id(0)
    p0 = lo_ref[i] // _TC_PAN
    p1 = hi_ref[i] // _TC_PAN
    idx = batch_ref[0, :, :]

    def panel(p, acc):
        g0 = p * _TC_PAN
        col = g0 + lax.broadcasted_iota(jnp.int32, (1, _TC_PAN), 1)
        oh = (idx == col).astype(jnp.bfloat16)
        rows = emb_ref[pl.ds(g0, _TC_PAN), :]
        return acc + jnp.dot(oh, rows, preferred_element_type=jnp.float32)

    out_ref[:, :] = lax.fori_loop(
        p0, p1 + 1, panel, jnp.zeros((_TC_BLK, _EMB), jnp.float32)
    )


def _tc_gather(emb_bf, batch, nblk):
    batch3 = batch.reshape(nblk, _TC_BLK, 1)
    lo = batch3[:, 0, 0]
    hi = batch3[:, _TC_BLK - 1, 0]
    grid_spec = pltpu.PrefetchScalarGridSpec(
        num_scalar_prefetch=2,
        grid=(nblk,),
        in_specs=[
            pl.BlockSpec((1, _TC_BLK, 1), lambda i, lo_r, hi_r: (i, 0, 0)),
            pl.BlockSpec((_NG, _EMB), lambda i, lo_r, hi_r: (0, 0)),
        ],
        out_specs=pl.BlockSpec((_TC_BLK, _EMB), lambda i, lo_r, hi_r: (i, 0)),
    )
    return pl.pallas_call(
        _tc_gather_body,
        grid_spec=grid_spec,
        out_shape=jax.ShapeDtypeStruct((nblk * _TC_BLK, _EMB), jnp.float32),
    )(lo, hi, batch3, emb_bf)


def kernel(t_sampled, batch):
    emb = _emb_table(t_sampled.astype(jnp.float32))
    return _tc_gather(emb.astype(jnp.bfloat16), batch, _NA // _TC_BLK)


# speedup vs baseline: 6.0876x; 6.0876x over previous
import functools

import jax
import jax.numpy as jnp
from jax import lax
from jax.experimental import pallas as pl
from jax.experimental.pallas import tpu as pltpu
from jax.experimental.pallas import tpu_sc as plsc

_EMB = 128
_HALF = 64
_NG = 8192
_NA = 524288

_NC = 2
_NS = 16
_NW = _NC * _NS

_CHUNK = 64
_ROWS_PER_W = _NA // _NW
_NCHUNKS = _ROWS_PER_W // _CHUNK
_LEAD = 2


def _emb_body(t_ref, out_ref):
    t = t_ref[:, :]
    k = lax.broadcasted_iota(jnp.int32, (1, _EMB), 1).astype(jnp.float32)
    kmod = jnp.where(k < _HALF, k, k - _HALF)
    freqs = jnp.exp(-jnp.log(10000.0) * kmod / (_HALF - 1))
    phase = jnp.where(k < _HALF, 0.0, jnp.pi / 2.0)
    x = t * freqs + phase
    n = jnp.round(x * (1.0 / (2.0 * jnp.pi)))
    r = (x - n * 6.28125) - n * 1.9353071795864769e-3
    r2 = r * r
    p = 1.6059043836e-10
    for c in (-2.5052108385e-8, 2.7557319224e-6, -1.984126984e-4,
              8.3333333333e-3, -1.6666666666e-1):
        p = p * r2 + c
    out_ref[:, :] = r + r * (r2 * p)


def _emb_table(t_sampled):
    return pl.pallas_call(
        _emb_body,
        out_shape=jax.ShapeDtypeStruct((_NG, _EMB), jnp.float32),
    )(t_sampled)


_NBUF = 4
assert _NCHUNKS % _NBUF == 0
assert 0 < _LEAD < _NBUF


def _sc_gather_body(emb_hbm, batch_hbm, out_hbm, tbl_sh, idx_v, rows_v, gsem, ssem):
    sid = lax.axis_index("s")
    wid = sid * _NC + lax.axis_index("c")
    base = wid * _ROWS_PER_W

    trows = _NG // _NS
    pltpu.sync_copy(
        emb_hbm.at[pl.ds(sid * trows, trows)],
        tbl_sh.at[pl.ds(sid * trows, trows)],
    )

    pltpu.sync_copy(batch_hbm.at[wid], idx_v)
    plsc.subcore_barrier()

    def start_gather(j, b):
        pltpu.async_copy(tbl_sh.at[idx_v.at[j]], rows_v.at[b], gsem.at[b])

    def wait_gather(b):
        pltpu.make_async_copy(
            emb_hbm.at[pl.ds(0, _CHUNK)], rows_v.at[b], gsem.at[b]
        ).wait()

    def start_scatter(j, b):
        pltpu.async_copy(
            rows_v.at[b], out_hbm.at[pl.ds(base + j * _CHUNK, _CHUNK)], ssem.at[b]
        )

    def wait_scatter(b):
        pltpu.make_async_copy(
            emb_hbm.at[pl.ds(0, _CHUNK)], rows_v.at[b], ssem.at[b]
        ).wait()

    for j in range(_LEAD):
        start_gather(j, j)

    n_outer = _NCHUNKS // _NBUF

    def outer(go, carry):
        for b in range(_NBUF):
            g = go * _NBUF + b
            wait_gather(b)
            start_scatter(g, b)
            bg = (b + _LEAD) % _NBUF

            if b >= _NBUF - _LEAD:
                wait_scatter(bg)
            else:

                @pl.when(go > 0)
                def _():
                    wait_scatter(bg)

            if b < _NBUF - _LEAD:
                start_gather(g + _LEAD, bg)
            else:

                @pl.when(go < n_outer - 1)
                def _():
                    start_gather(g + _LEAD, bg)

        return carry

    lax.fori_loop(0, n_outer, outer, 0)

    for k in range(_NBUF - _LEAD):
        wait_scatter((_NCHUNKS - _NBUF + _LEAD + k) % _NBUF)


_sc_gather = functools.partial(
    pl.kernel,
    mesh=plsc.VectorSubcoreMesh(core_axis_name="c", subcore_axis_name="s"),
    out_type=jax.ShapeDtypeStruct((_NA, _EMB), jnp.float32),
    scratch_types=[
        pltpu.VMEM_SHARED((_NG, _EMB), jnp.float32),
        pltpu.VMEM((_NCHUNKS, _CHUNK), jnp.int32),
        pltpu.VMEM((_NBUF, _CHUNK, _EMB), jnp.float32),
        pltpu.SemaphoreType.DMA((_NBUF,)),
        pltpu.SemaphoreType.DMA((_NBUF,)),
    ],
)(_sc_gather_body)


def kernel(t_sampled, batch):
    emb = _emb_table(t_sampled.astype(jnp.float32))
    batch_r = batch.reshape(_NW, _NCHUNKS, _CHUNK)
    return _sc_gather(emb, batch_r)
